# baseline (speedup 1.0000x reference)
import functools

import jax
import jax.numpy as jnp
from jax import lax
from jax.experimental import pallas as pl
from jax.experimental.pallas import tpu as pltpu
from jax.experimental.pallas import tpu_sc as plsc

_N = 320000
_D = 128
_C = 4
_L = 16
_NC = 2
_NS = 16
_NW = _NC * _NS
_ROWS_W = _N // _NW
_CH = 80
_NCHUNK = _ROWS_W // _CH


def _sc_partials(features, labels_i32):
    mesh = plsc.VectorSubcoreMesh(core_axis_name="c", subcore_axis_name="s")

    @functools.partial(
        pl.kernel,
        out_type=[
            jax.ShapeDtypeStruct((_NW * _C, _D), jnp.float32),
            jax.ShapeDtypeStruct((_NW * _C, _L), jnp.float32),
        ],
        mesh=mesh,
        scratch_types=[
            pltpu.VMEM((_CH, _D), jnp.float32),
            pltpu.VMEM((_CH,), jnp.int32),
            pltpu.VMEM((_CH,), jnp.int32),
            pltpu.VMEM((_C, _D), jnp.float32),
            pltpu.VMEM((_C, _L), jnp.float32),
            pltpu.VMEM_SHARED((_NS * _C, _D), jnp.float32),
        ],
    )
    def k(feat_hbm, lab_hbm, sums_hbm, cnt_hbm, fbuf, lbuf, lbuf_off, zbuf,
          cnt, shared_acc):
        cid = lax.axis_index("c")
        sid = lax.axis_index("s")
        wid = cid * _NS + sid
        base = wid * _ROWS_W
        bank = sid * _C
        zero = jnp.zeros((_L,), jnp.float32)
        for r in range(_C):
            for j in range(_D // _L):
                zbuf[r, pl.ds(j * _L, _L)] = zero
            cnt[r, :] = zero
        pltpu.sync_copy(zbuf, shared_acc.at[pl.ds(bank, _C)])

        def step(i, carry):
            row0 = base + i * _CH
            pltpu.sync_copy(lab_hbm.at[pl.ds(row0, _CH)], lbuf)
            pltpu.sync_copy(feat_hbm.at[pl.ds(row0, _CH)], fbuf)
            for g in range(_CH // _L):
                sl = pl.ds(g * _L, _L)
                lv = lbuf[sl]
                lbuf_off[sl] = lv + bank
                for r in range(_C):
                    cnt[r, :] += jnp.where(lv == r, 1.0, 0.0)
            pltpu.sync_copy(fbuf, shared_acc.at[lbuf_off], add=True)
            return carry

        lax.fori_loop(0, _NCHUNK, step, 0)
        pltpu.sync_copy(shared_acc.at[pl.ds(bank, _C)],
                        sums_hbm.at[pl.ds(wid * _C, _C)])
        pltpu.sync_copy(cnt, cnt_hbm.at[pl.ds(wid * _C, _C)])

    return k(features, labels_i32)


def _tc_body(s_ref, c_ref, out_ref):
    tot = s_ref[0:_C, :]
    for w in range(1, _NW):
        tot += s_ref[w * _C:(w + 1) * _C, :]
    ctot = c_ref[0:_C, :]
    for w in range(1, _NW):
        ctot += c_ref[w * _C:(w + 1) * _C, :]
    for cl in range(_C):
        n_cl = jnp.sum(ctot[cl, :])
        mean = tot[cl, :] / jnp.maximum(n_cl, 1.0)
        nrm = jnp.sqrt(jnp.sum(mean * mean))
        out_ref[cl, :] = mean / jnp.maximum(nrm, 1e-12)


def _tc_finalize(sums, cnts):
    return pl.pallas_call(
        _tc_body,
        out_shape=jax.ShapeDtypeStruct((_C, _D), jnp.float32),
    )(sums, cnts)


def kernel(features, labels):
    sums, cnts = _sc_partials(features, labels.astype(jnp.int32))
    fea_center = _tc_finalize(sums, cnts)
    target = jnp.array([0, 1, 2, 3], dtype=jnp.int64)
    return (fea_center, target)


# speedup vs baseline: 2.1397x; 2.1397x over previous
import functools

import jax
import jax.numpy as jnp
from jax import lax
from jax.experimental import pallas as pl
from jax.experimental.pallas import tpu as pltpu
from jax.experimental.pallas import tpu_sc as plsc

_N = 320000
_D = 128
_C = 4
_L = 16
_NC = 2
_NS = 16
_NW = _NC * _NS
_ROWS_W = _N // _NW
_CH = 80
_NCHUNK = _ROWS_W // _CH


def _sc_partials(features, labels_i32):
    mesh = plsc.VectorSubcoreMesh(core_axis_name="c", subcore_axis_name="s")

    @functools.partial(
        pl.kernel,
        out_type=[
            jax.ShapeDtypeStruct((_NW * _C, _D), jnp.float32),
            jax.ShapeDtypeStruct((_NW * _C, _L), jnp.float32),
        ],
        mesh=mesh,
        scratch_types=[
            pltpu.VMEM((2, _CH, _D), jnp.float32),
            pltpu.VMEM((2, _CH), jnp.int32),
            pltpu.VMEM((2, _CH), jnp.int32),
            pltpu.VMEM((_C, _D), jnp.float32),
            pltpu.VMEM((_C, _L), jnp.float32),
            pltpu.VMEM_SHARED((_NS * _C, _D), jnp.float32),
            pltpu.SemaphoreType.DMA,
            pltpu.SemaphoreType.DMA,
            pltpu.SemaphoreType.DMA,
            pltpu.SemaphoreType.DMA,
        ],
    )
    def k(feat_hbm, lab_hbm, sums_hbm, cnt_hbm, fbuf, lbuf, lidx, zbuf, cnt,
          shared_acc, isem0, isem1, ssem0, ssem1):
        cid = lax.axis_index("c")
        sid = lax.axis_index("s")
        wid = cid * _NS + sid
        base = wid * _ROWS_W
        bank = sid * _C
        isem = (isem0, isem1)
        ssem = (ssem0, ssem1)
        zero = jnp.zeros((_L,), jnp.float32)
        for r in range(_C):
            for j in range(_D // _L):
                zbuf[r, pl.ds(j * _L, _L)] = zero
            cnt[r, :] = zero
        pltpu.sync_copy(zbuf, shared_acc.at[pl.ds(bank, _C)])

        def issue_inputs(i, buf):
            row0 = base + i * _CH
            pltpu.async_copy(lab_hbm.at[pl.ds(row0, _CH)], lbuf.at[buf],
                             isem[buf])
            pltpu.async_copy(feat_hbm.at[pl.ds(row0, _CH)], fbuf.at[buf],
                             isem[buf])

        def wait_inputs(i, buf):
            row0 = base + i * _CH
            pltpu.make_async_copy(lab_hbm.at[pl.ds(row0, _CH)], lbuf.at[buf],
                                  isem[buf]).wait()
            pltpu.make_async_copy(feat_hbm.at[pl.ds(row0, _CH)], fbuf.at[buf],
                                  isem[buf]).wait()

        def compute(buf):
            for g in range(_CH // _L):
                sl = pl.ds(g * _L, _L)
                lv = lbuf[buf, sl]
                lidx[buf, sl] = lv + bank
                for r in range(_C):
                    cnt[r, :] += jnp.where(lv == r, 1.0, 0.0)

        def issue_scatter(buf):
            pltpu.async_copy(fbuf.at[buf], shared_acc.at[lidx.at[buf]],
                             ssem[buf], add=True)

        def wait_scatter(buf):
            pltpu.make_async_copy(fbuf.at[buf], shared_acc.at[lidx.at[buf]],
                                  ssem[buf]).wait()

        issue_inputs(0, 0)
        issue_inputs(1, 1)
        wait_inputs(0, 0)
        compute(0)
        issue_scatter(0)

        def body(i2, carry):
            for b in range(2):
                i = 1 + 2 * i2 + b
                buf = (1 + b) % 2
                wait_scatter(1 - buf)
                issue_inputs(i + 1, 1 - buf)
                wait_inputs(i, buf)
                compute(buf)
                issue_scatter(buf)
            return carry

        lax.fori_loop(0, (_NCHUNK - 3) // 2, body, 0)

        wait_scatter(0)
        issue_inputs(_NCHUNK - 1, 0)
        wait_inputs(_NCHUNK - 2, 1)
        compute(1)
        issue_scatter(1)
        wait_scatter(1)
        wait_inputs(_NCHUNK - 1, 0)
        compute(0)
        issue_scatter(0)
        wait_scatter(0)

        pltpu.sync_copy(shared_acc.at[pl.ds(bank, _C)],
                        sums_hbm.at[pl.ds(wid * _C, _C)])
        pltpu.sync_copy(cnt, cnt_hbm.at[pl.ds(wid * _C, _C)])

    return k(features, labels_i32)


def _tc_body(s_ref, c_ref, out_ref):
    tot = s_ref[0:_C, :]
    for w in range(1, _NW):
        tot += s_ref[w * _C:(w + 1) * _C, :]
    ctot = c_ref[0:_C, :]
    for w in range(1, _NW):
        ctot += c_ref[w * _C:(w + 1) * _C, :]
    for cl in range(_C):
        n_cl = jnp.sum(ctot[cl, :])
        mean = tot[cl, :] / jnp.maximum(n_cl, 1.0)
        nrm = jnp.sqrt(jnp.sum(mean * mean))
        out_ref[cl, :] = mean / jnp.maximum(nrm, 1e-12)


def _tc_finalize(sums, cnts):
    return pl.pallas_call(
        _tc_body,
        out_shape=jax.ShapeDtypeStruct((_C, _D), jnp.float32),
    )(sums, cnts)


def kernel(features, labels):
    sums, cnts = _sc_partials(features, labels.astype(jnp.int32))
    fea_center = _tc_finalize(sums, cnts)
    target = jnp.array([0, 1, 2, 3], dtype=jnp.int64)
    return (fea_center, target)
